# Initial kernel scaffold; baseline (speedup 1.0000x reference)
import functools

import jax
import jax.numpy as jnp
from jax import lax
from jax.experimental import pallas as pl
from jax.experimental.pallas import tpu as pltpu
from jax.experimental.pallas import tpu_sc as plsc

VOCAB = 1000000
EMBED_DIM = 32

_INFO = plsc.get_sparse_core_info()
_NC, _NS = _INFO.num_cores, _INFO.num_subcores
_NW = _NC * _NS

_CHUNK = 128


def _emb_kernel(n_chunks: int, table_hbm, idx_hbm, out_hbm, idx_v, rows_v, sem):
    wid = lax.axis_index("s") * _NC + lax.axis_index("c")
    base = wid * (n_chunks * _CHUNK)
    pltpu.sync_copy(idx_hbm.at[wid], idx_v)

    def body(j, _):
        pltpu.async_copy(table_hbm.at[idx_v.at[j]], rows_v, sem).wait()
        pltpu.sync_copy(rows_v, out_hbm.at[pl.ds(base + j * _CHUNK, _CHUNK)])
        return 0

    lax.fori_loop(0, n_chunks, body, 0)


def kernel(x, table):
    orig_shape = x.shape
    flat = x.reshape(-1).astype(jnp.int32)
    n = flat.shape[0]
    assert n % (_NW * _CHUNK) == 0, n
    n_chunks = n // (_NW * _CHUNK)
    idx = flat.reshape(_NW, n_chunks, _CHUNK)

    k = functools.partial(
        pl.kernel,
        mesh=plsc.VectorSubcoreMesh(core_axis_name="c", subcore_axis_name="s"),
        out_type=jax.ShapeDtypeStruct((n, EMBED_DIM), jnp.float32),
        scratch_types=[
            pltpu.VMEM((n_chunks, _CHUNK), jnp.int32),
            pltpu.VMEM((_CHUNK, EMBED_DIM), jnp.float32),
            pltpu.SemaphoreType.DMA,
        ],
    )(functools.partial(_emb_kernel, n_chunks))

    out = k(table, idx)
    return out.reshape(*orig_shape, EMBED_DIM)


# speedup vs baseline: 1.3061x; 1.3061x over previous
import functools

import jax
import jax.numpy as jnp
from jax import lax
from jax.experimental import pallas as pl
from jax.experimental.pallas import tpu as pltpu
from jax.experimental.pallas import tpu_sc as plsc

VOCAB = 1000000
EMBED_DIM = 32

_INFO = plsc.get_sparse_core_info()
_NC, _NS = _INFO.num_cores, _INFO.num_subcores
_NW = _NC * _NS

_CHUNK = 128


def _emb_kernel(n_chunks: int, table_hbm, idx_hbm, out_hbm, idx_v, rows_v, sem):
    wid = lax.axis_index("s") * _NC + lax.axis_index("c")
    base = wid * (n_chunks * _CHUNK)
    pltpu.sync_copy(idx_hbm.at[wid], idx_v)

    def body(j, _):
        pltpu.async_copy(table_hbm.at[idx_v.at[j]], rows_v, sem).wait()
        pltpu.sync_copy(rows_v, out_hbm.at[pl.ds(base + j * _CHUNK, _CHUNK)])
        return 0

    lax.fori_loop(0, n_chunks, body, 0)


def kernel(x, table):
    orig_shape = x.shape
    flat = x.reshape(-1).astype(jnp.int32)
    n = flat.shape[0]
    assert n % (_NW * _CHUNK) == 0, n
    n_chunks = n // (_NW * _CHUNK)
    idx = flat.reshape(_NW, n_chunks, _CHUNK)

    k = functools.partial(
        pl.kernel,
        mesh=plsc.VectorSubcoreMesh(core_axis_name="c", subcore_axis_name="s"),
        out_type=jax.ShapeDtypeStruct((n, EMBED_DIM), jnp.float32),
        scratch_types=[
            pltpu.VMEM((n_chunks, _CHUNK), jnp.int32),
            pltpu.VMEM((_CHUNK, EMBED_DIM), jnp.float32),
            pltpu.SemaphoreType.DMA,
        ],
        compiler_params=pltpu.CompilerParams(use_tc_tiling_on_sc=False),
    )(functools.partial(_emb_kernel, n_chunks))

    out = k(table, idx)
    return out.reshape(*orig_shape, EMBED_DIM)


# speedup vs baseline: 1.4995x; 1.1481x over previous
import functools

import jax
import jax.numpy as jnp
from jax import lax
from jax.experimental import pallas as pl
from jax.experimental.pallas import tpu as pltpu
from jax.experimental.pallas import tpu_sc as plsc

VOCAB = 1000000
EMBED_DIM = 32

_INFO = plsc.get_sparse_core_info()
_NC, _NS = _INFO.num_cores, _INFO.num_subcores
_NW = _NC * _NS

_CHUNK = 128
_G = 10
_SLAB = _G * _CHUNK


def _emb_kernel(n_chunks: int, table_hbm, idx_hbm, out_hbm,
                idx_v, rows_v, g_sem, st_sem0, st_sem1):
    n_slabs = n_chunks // _G
    wid = lax.axis_index("s") * _NC + lax.axis_index("c")
    base = wid * (n_chunks * _CHUNK)
    st_sems = (st_sem0, st_sem1)
    pltpu.sync_copy(idx_hbm.at[wid], idx_v)

    def do_slab(t, b):
        cps = [
            pltpu.async_copy(
                table_hbm.at[idx_v.at[t * _G + g]],
                rows_v.at[b, pl.ds(g * _CHUNK, _CHUNK)],
                g_sem,
            )
            for g in range(_G)
        ]
        for cp in cps:
            cp.wait()
        pltpu.async_copy(
            rows_v.at[b],
            out_hbm.at[pl.ds(base + t * _SLAB, _SLAB)],
            st_sems[b],
        )

    def wait_store(b):
        pltpu.make_async_copy(
            rows_v.at[b], out_hbm.at[pl.ds(base, _SLAB)], st_sems[b]
        ).wait()

    def body(i, _):
        for b in range(2):
            @pl.when(i >= 1)
            def _():
                wait_store(b)
            do_slab(i * 2 + b, b)
        return 0

    lax.fori_loop(0, n_slabs // 2, body, 0)
    wait_store(0)
    wait_store(1)


def kernel(x, table):
    orig_shape = x.shape
    flat = x.reshape(-1).astype(jnp.int32)
    n = flat.shape[0]
    assert n % (_NW * _SLAB * 2) == 0, n
    n_chunks = n // (_NW * _CHUNK)
    idx = flat.reshape(_NW, n_chunks, _CHUNK)

    k = functools.partial(
        pl.kernel,
        mesh=plsc.VectorSubcoreMesh(core_axis_name="c", subcore_axis_name="s"),
        out_type=jax.ShapeDtypeStruct((n, EMBED_DIM), jnp.float32),
        scratch_types=[
            pltpu.VMEM((n_chunks, _CHUNK), jnp.int32),
            pltpu.VMEM((2, _SLAB, EMBED_DIM), jnp.float32),
            pltpu.SemaphoreType.DMA,
            pltpu.SemaphoreType.DMA,
            pltpu.SemaphoreType.DMA,
        ],
        compiler_params=pltpu.CompilerParams(use_tc_tiling_on_sc=False),
    )(functools.partial(_emb_kernel, n_chunks))

    out = k(table, idx)
    return out.reshape(*orig_shape, EMBED_DIM)
